# baseline (speedup 1.0000x reference)
import functools
import math

import jax
import jax.numpy as jnp
from jax import lax
from jax.experimental import pallas as pl
from jax.experimental.pallas import tpu as pltpu
from jax.experimental.pallas import tpu_sc as plsc

B, C, L, D = 4096, 20, 50, 64
MP = 64
CP = 32
CO = 32
NC, NS = 2, 16
NW = NC * NS
BW = B // NW
NV = D // 16
G = 4
NCH = BW // G
NB = 2
DP = 2 * D

_GDN = lax.GatherDimensionNumbers(
    offset_dims=(), collapsed_slice_dims=(0,), start_index_map=(0,))


def _permute(v, idx):
    return lax.gather(v, idx[:, None], dimension_numbers=_GDN,
                      slice_sizes=(1,),
                      mode=lax.GatherScatterMode.PROMISE_IN_BOUNDS)


def _lanesum(v, perms):
    for p in perms:
        v = v + _permute(v, p)
    return v


def _body(emb_hbm, cdd_hbm, his_hbm, mask_hbm,
          out_hbm, cdd_idx_v, his_idx_v, mask_v,
          logits_v, his_rows, cdd_rows, sems_h, sems_c):
    wid = lax.axis_index("s") * NC + lax.axis_index("c")
    base = wid * BW

    pltpu.sync_copy(cdd_hbm.at[pl.ds(base * C, BW * C)], cdd_idx_v)
    pltpu.sync_copy(his_hbm.at[pl.ds(base * L, BW * L)], his_idx_v)
    pltpu.sync_copy(mask_hbm.at[pl.ds(base * MP, BW * MP)], mask_v)

    lane = lax.iota(jnp.int32, 16)
    perms = [lane ^ k for k in (1, 2, 4, 8)]

    def copies(ci, slot):
        cic = jnp.minimum(ci, NCH - 1)
        h = pltpu.make_async_copy(
            emb_hbm.at[his_idx_v.at[pl.ds(cic * (G * L), G * L)]],
            his_rows.at[slot], sems_h[slot])
        c = pltpu.make_async_copy(
            emb_hbm.at[cdd_idx_v.at[pl.ds(cic * (G * C), G * C)]],
            cdd_rows.at[slot], sems_c[slot])
        return h, c

    for s in range(NB):
        h, c = copies(jnp.int32(s), s)
        h.start()
        c.start()

    def super_body(gi, _):
        for s in range(NB):
            ci = gi * NB + s
            h, c = copies(ci, s)
            h.wait()
            c.wait()

            def batch_body(bq, _):
                bi = ci * G + bq

                mvecs = [mask_v[pl.ds(bi * MP + 16 * q, 16)]
                         for q in range(MP // 16)]
                msum_vec = mvecs[0]
                for q in range(1, MP // 16):
                    msum_vec = msum_vec + mvecs[q]
                inv = 1.0 / (_lanesum(msum_vec, perms) + 1e-6)

                acc = [jnp.zeros((16,), jnp.float32) for _ in range(NV)]
                for l in range(L):
                    m = mvecs[l // 16][l % 16]
                    for j in range(NV):
                        acc[j] = acc[j] + m * his_rows[
                            s, bq * L + l, pl.ds(16 * j, 16)]
                scale = inv * (1.0 / math.sqrt(D))
                user = [acc[j] * scale for j in range(NV)]

                rows = [jnp.zeros((16,), jnp.float32)
                        for _ in range(CO // 16)]
                for cc in range(C):
                    dot = cdd_rows[s, bq * C + cc,
                                   pl.ds(0, 16)] * user[0]
                    for j in range(1, NV):
                        dot = dot + (cdd_rows[
                            s, bq * C + cc,
                            pl.ds(16 * j, 16)] * user[j])
                    sv = _lanesum(dot, perms)
                    rows[cc // 16] = jnp.where(lane == (cc % 16), sv,
                                               rows[cc // 16])
                for q in range(CO // 16):
                    logits_v[pl.ds(bi * CO + 16 * q, 16)] = rows[q]
                return ()

            lax.fori_loop(0, G, batch_body, ())

            h2, c2 = copies(ci + NB, s)
            h2.start()
            c2.start()
        return ()

    lax.fori_loop(0, NCH // NB, super_body, ())

    for s in range(NB):
        h, c = copies(jnp.int32(NCH - 1), s)
        h.wait()
        c.wait()

    pltpu.sync_copy(logits_v, out_hbm.at[pl.ds(base * CO, BW * CO)])


@functools.partial(
    pl.kernel,
    out_type=jax.ShapeDtypeStruct((B * CO,), jnp.float32),
    mesh=plsc.VectorSubcoreMesh(core_axis_name="c", subcore_axis_name="s"),
    compiler_params=pltpu.CompilerParams(use_tc_tiling_on_sc=True),
    scratch_types=[
        pltpu.VMEM((BW * C,), jnp.int32),
        pltpu.VMEM((BW * L,), jnp.int32),
        pltpu.VMEM((BW * MP,), jnp.float32),
        pltpu.VMEM((BW * CO,), jnp.float32),
        pltpu.VMEM((NB, G * L, DP), jnp.float32),
        pltpu.VMEM((NB, G * C, DP), jnp.float32),
        [pltpu.SemaphoreType.DMA] * NB,
        [pltpu.SemaphoreType.DMA] * NB,
    ],
)
def _sc_two_tower(*args):
    _body(*args)


BR = 8192


def _pad_block(in_ref, out_ref):
    out_ref[:, :D] = in_ref[...]
    out_ref[:, D:] = jnp.zeros((BR, D), jnp.float32)


_widen_table = pl.pallas_call(
    _pad_block,
    grid=(1000000 // BR,),
    in_specs=[pl.BlockSpec((BR, D), lambda i: (i, 0))],
    out_specs=pl.BlockSpec((BR, DP), lambda i: (i, 0)),
    out_shape=jax.ShapeDtypeStruct((1000000, DP), jnp.float32),
)


def kernel(news_embeddings, cdd_idx, his_idx, his_mask):
    emb_wide = _widen_table(news_embeddings)
    cdd_f = cdd_idx.astype(jnp.int32).reshape(B * C)
    his_f = his_idx.astype(jnp.int32).reshape(B * L)
    mask_f = jnp.pad(his_mask, ((0, 0), (0, MP - L))).reshape(B * MP)
    out = _sc_two_tower(emb_wide, cdd_f, his_f, mask_f)
    return out.reshape(B, CO)[:, :C]


# speedup vs baseline: 1.0001x; 1.0001x over previous
import functools
import math

import jax
import jax.numpy as jnp
from jax import lax
from jax.experimental import pallas as pl
from jax.experimental.pallas import tpu as pltpu
from jax.experimental.pallas import tpu_sc as plsc

B, C, L, D = 4096, 20, 50, 64
MP = 64
CP = 32
CO = 32
NC, NS = 2, 16
NW = NC * NS
BW = B // NW
NV = D // 16
G = 4
NCH = BW // G
NB = 2
DP = 2 * D

_GDN = lax.GatherDimensionNumbers(
    offset_dims=(), collapsed_slice_dims=(0,), start_index_map=(0,))


def _permute(v, idx):
    return lax.gather(v, idx[:, None], dimension_numbers=_GDN,
                      slice_sizes=(1,),
                      mode=lax.GatherScatterMode.PROMISE_IN_BOUNDS)


def _lanesum(v, perms):
    for p in perms:
        v = v + _permute(v, p)
    return v


def _body(emb_hbm, cdd_hbm, his_hbm, mask_hbm,
          out_hbm, cdd_idx_v, his_idx_v, mask_v,
          logits_v, his_rows, cdd_rows, sems_h, sems_c):
    wid = lax.axis_index("s") * NC + lax.axis_index("c")
    base = wid * BW

    pltpu.sync_copy(cdd_hbm.at[pl.ds(base * C, BW * C)], cdd_idx_v)
    pltpu.sync_copy(his_hbm.at[pl.ds(base * L, BW * L)], his_idx_v)
    pltpu.sync_copy(mask_hbm.at[pl.ds(base * MP, BW * MP)], mask_v)

    lane = lax.iota(jnp.int32, 16)
    perms = [lane ^ k for k in (1, 2, 4, 8)]

    def copies(ci, slot):
        cic = jnp.minimum(ci, NCH - 1)
        h = pltpu.make_async_copy(
            emb_hbm.at[his_idx_v.at[pl.ds(cic * (G * L), G * L)]],
            his_rows.at[slot], sems_h[slot])
        c = pltpu.make_async_copy(
            emb_hbm.at[cdd_idx_v.at[pl.ds(cic * (G * C), G * C)]],
            cdd_rows.at[slot], sems_c[slot])
        return h, c

    for s in range(NB):
        h, c = copies(jnp.int32(s), s)
        h.start()
        c.start()

    def super_body(gi, _):
        for s in range(NB):
            ci = gi * NB + s
            h, c = copies(ci, s)
            h.wait()
            c.wait()

            def batch_body(bq, _):
                bi = ci * G + bq

                mvecs = [mask_v[pl.ds(bi * MP + 16 * q, 16)]
                         for q in range(MP // 16)]
                msum_vec = mvecs[0]
                for q in range(1, MP // 16):
                    msum_vec = msum_vec + mvecs[q]
                inv = 1.0 / (_lanesum(msum_vec, perms) + 1e-6)

                acc = [jnp.zeros((16,), jnp.float32) for _ in range(NV)]
                for l in range(L):
                    m = mvecs[l // 16][l % 16]
                    for j in range(NV):
                        acc[j] = acc[j] + m * his_rows[
                            s, bq * L + l, pl.ds(16 * j, 16)]
                scale = inv * (1.0 / math.sqrt(D))
                user = [acc[j] * scale for j in range(NV)]

                rows = [jnp.zeros((16,), jnp.float32)
                        for _ in range(CO // 16)]
                for cc in range(C):
                    dot = cdd_rows[s, bq * C + cc,
                                   pl.ds(0, 16)] * user[0]
                    for j in range(1, NV):
                        dot = dot + (cdd_rows[
                            s, bq * C + cc,
                            pl.ds(16 * j, 16)] * user[j])
                    sv = _lanesum(dot, perms)
                    rows[cc // 16] = jnp.where(lane == (cc % 16), sv,
                                               rows[cc // 16])
                for q in range(CO // 16):
                    logits_v[pl.ds(bi * CO + 16 * q, 16)] = rows[q]
                return ()

            lax.fori_loop(0, G, batch_body, ())

            h2, c2 = copies(ci + NB, s)
            h2.start()
            c2.start()
        return ()

    lax.fori_loop(0, NCH // NB, super_body, ())

    for s in range(NB):
        h, c = copies(jnp.int32(NCH - 1), s)
        h.wait()
        c.wait()

    pltpu.sync_copy(logits_v, out_hbm.at[pl.ds(base * CO, BW * CO)])


@functools.partial(
    pl.kernel,
    out_type=jax.ShapeDtypeStruct((B * CO,), jnp.float32),
    mesh=plsc.VectorSubcoreMesh(core_axis_name="c", subcore_axis_name="s"),
    compiler_params=pltpu.CompilerParams(use_tc_tiling_on_sc=True),
    scratch_types=[
        pltpu.VMEM((BW * C,), jnp.int32),
        pltpu.VMEM((BW * L,), jnp.int32),
        pltpu.VMEM((BW * MP,), jnp.float32),
        pltpu.VMEM((BW * CO,), jnp.float32),
        pltpu.VMEM((NB, G * L, DP), jnp.float32),
        pltpu.VMEM((NB, G * C, DP), jnp.float32),
        [pltpu.SemaphoreType.DMA] * NB,
        [pltpu.SemaphoreType.DMA] * NB,
    ],
)
def _sc_two_tower(*args):
    _body(*args)


BR = 8000


def _pad_block(in_ref, out_ref):
    out_ref[:, :D] = in_ref[...]
    out_ref[:, D:] = jnp.zeros((BR, D), jnp.float32)


_widen_table = pl.pallas_call(
    _pad_block,
    grid=(1000000 // BR,),
    in_specs=[pl.BlockSpec((BR, D), lambda i: (i, 0))],
    out_specs=pl.BlockSpec((BR, DP), lambda i: (i, 0)),
    out_shape=jax.ShapeDtypeStruct((1000000, DP), jnp.float32),
)


def kernel(news_embeddings, cdd_idx, his_idx, his_mask):
    emb_wide = _widen_table(news_embeddings)
    cdd_f = cdd_idx.astype(jnp.int32).reshape(B * C)
    his_f = his_idx.astype(jnp.int32).reshape(B * L)
    mask_f = jnp.pad(his_mask, ((0, 0), (0, MP - L))).reshape(B * MP)
    out = _sc_two_tower(emb_wide, cdd_f, his_f, mask_f)
    return out.reshape(B, CO)[:, :C]


# speedup vs baseline: 1.2129x; 1.2127x over previous
import functools
import math

import jax
import jax.numpy as jnp
from jax import lax
from jax.experimental import pallas as pl
from jax.experimental.pallas import tpu as pltpu
from jax.experimental.pallas import tpu_sc as plsc

B, C, L, D = 4096, 20, 50, 64
MP = 64
CP = 32
CO = 32
NC, NS = 2, 16
NW = NC * NS
BW = B // NW
NV = D // 16
G = 4
NCH = BW // G
NB = 2
DP = 2 * D

_GDN = lax.GatherDimensionNumbers(
    offset_dims=(), collapsed_slice_dims=(0,), start_index_map=(0,))


def _permute(v, idx):
    return lax.gather(v, idx[:, None], dimension_numbers=_GDN,
                      slice_sizes=(1,),
                      mode=lax.GatherScatterMode.PROMISE_IN_BOUNDS)


def _lanesum(v, perms):
    for p in perms:
        v = v + _permute(v, p)
    return v


def _body(emb_hbm, cdd_hbm, his_hbm, mask_hbm,
          out_hbm, cdd_idx_v, his_idx_v, mask_v,
          logits_v, his_rows, cdd_rows, sems_h, sems_c):
    wid = lax.axis_index("s") * NC + lax.axis_index("c")
    base = wid * BW

    pltpu.sync_copy(cdd_hbm.at[pl.ds(base * C, BW * C)], cdd_idx_v)
    pltpu.sync_copy(his_hbm.at[pl.ds(base * L, BW * L)], his_idx_v)
    pltpu.sync_copy(mask_hbm.at[pl.ds(base * MP, BW * MP)], mask_v)

    lane = lax.iota(jnp.int32, 16)
    perms = [lane ^ k for k in (1, 2, 4, 8)]

    def copies(ci, slot):
        cic = jnp.minimum(ci, NCH - 1)
        h = pltpu.make_async_copy(
            emb_hbm.at[his_idx_v.at[pl.ds(cic * (G * L), G * L)]],
            his_rows.at[slot], sems_h[slot])
        c = pltpu.make_async_copy(
            emb_hbm.at[cdd_idx_v.at[pl.ds(cic * (G * C), G * C)]],
            cdd_rows.at[slot], sems_c[slot])
        return h, c

    for s in range(NB):
        h, c = copies(jnp.int32(s), s)
        h.start()
        c.start()

    def super_body(gi, _):
        for s in range(NB):
            ci = gi * NB + s
            h, c = copies(ci, s)
            h.wait()
            c.wait()

            def batch_body(bq, _):
                bi = ci * G + bq

                mvecs = [mask_v[pl.ds(bi * MP + 16 * q, 16)]
                         for q in range(MP // 16)]
                msum_vec = mvecs[0]
                for q in range(1, MP // 16):
                    msum_vec = msum_vec + mvecs[q]
                inv = 1.0 / (_lanesum(msum_vec, perms) + 1e-6)

                acc = [jnp.zeros((16,), jnp.float32) for _ in range(NV)]
                for l in range(L):
                    m = mvecs[l // 16][l % 16]
                    for j in range(NV):
                        acc[j] = acc[j] + m * his_rows[
                            s, bq * L + l, pl.ds(16 * j, 16)]
                scale = inv * (1.0 / math.sqrt(D))
                user = [acc[j] * scale for j in range(NV)]

                rows = [jnp.zeros((16,), jnp.float32)
                        for _ in range(CO // 16)]
                for cc in range(C):
                    dot = cdd_rows[s, bq * C + cc,
                                   pl.ds(0, 16)] * user[0]
                    for j in range(1, NV):
                        dot = dot + (cdd_rows[
                            s, bq * C + cc,
                            pl.ds(16 * j, 16)] * user[j])
                    sv = _lanesum(dot, perms)
                    rows[cc // 16] = jnp.where(lane == (cc % 16), sv,
                                               rows[cc // 16])
                for q in range(CO // 16):
                    logits_v[pl.ds(bi * CO + 16 * q, 16)] = rows[q]
                return ()

            lax.fori_loop(0, G, batch_body, ())

            h2, c2 = copies(ci + NB, s)
            h2.start()
            c2.start()
        return ()

    lax.fori_loop(0, NCH // NB, super_body, ())

    for s in range(NB):
        h, c = copies(jnp.int32(NCH - 1), s)
        h.wait()
        c.wait()

    pltpu.sync_copy(logits_v, out_hbm.at[pl.ds(base * CO, BW * CO)])


@functools.partial(
    pl.kernel,
    out_type=jax.ShapeDtypeStruct((B * CO,), jnp.float32),
    mesh=plsc.VectorSubcoreMesh(core_axis_name="c", subcore_axis_name="s"),
    compiler_params=pltpu.CompilerParams(use_tc_tiling_on_sc=True),
    scratch_types=[
        pltpu.VMEM((BW * C,), jnp.int32),
        pltpu.VMEM((BW * L,), jnp.int32),
        pltpu.VMEM((BW * MP,), jnp.float32),
        pltpu.VMEM((BW * CO,), jnp.float32),
        pltpu.VMEM((NB, G * L, DP), jnp.float32),
        pltpu.VMEM((NB, G * C, DP), jnp.float32),
        [pltpu.SemaphoreType.DMA] * NB,
        [pltpu.SemaphoreType.DMA] * NB,
    ],
)
def _sc_two_tower(*args):
    _body(*args)


def kernel(news_embeddings, cdd_idx, his_idx, his_mask):
    emb_wide = jnp.pad(news_embeddings, ((0, 0), (0, D)))
    cdd_f = cdd_idx.astype(jnp.int32).reshape(B * C)
    his_f = his_idx.astype(jnp.int32).reshape(B * L)
    mask_f = jnp.pad(his_mask, ((0, 0), (0, MP - L))).reshape(B * MP)
    out = _sc_two_tower(emb_wide, cdd_f, his_f, mask_f)
    return out.reshape(B, CO)[:, :C]
